# Initial kernel scaffold; baseline (speedup 1.0000x reference)
import functools
import math

import jax
import jax.numpy as jnp
from jax import lax
from jax.experimental import pallas as pl
from jax.experimental.pallas import tpu as pltpu

B, C, H, W = 2, 96, 224, 224
M = 12
K = 7
N = H * W

CB = 8
TB3 = 3584
TB4 = 3584


def _normalize(v, axis=-1):
    n = jnp.linalg.norm(v, axis=axis, keepdims=True)
    return v / jnp.maximum(n, 1e-12)


def _conv_kernel(x_ref, w_ref, b_ref, f_ref, pool_ref):
    xb = x_ref[...]
    acc = jnp.zeros((CB, H, W), jnp.float32)
    for dy in range(K):
        for dx in range(K):
            k = dy * K + dx
            wch = w_ref[:, k].reshape(CB, 1, 1)
            acc = acc + wch * xb[:, dy:dy + H, dx:dx + W]
    acc = acc + b_ref[:, 0].reshape(CB, 1, 1)
    f_ref[...] = acc
    pool_ref[...] = jnp.sum(acc, axis=(1, 2))[:, None]


def _run_conv(xpad, w49, bias):
    nblk = B * C // CB
    f, pool = pl.pallas_call(
        _conv_kernel,
        grid=(nblk,),
        in_specs=[
            pl.BlockSpec((CB, 232, 232), lambda i: (i, 0, 0)),
            pl.BlockSpec((CB, K * K), lambda i: (i % (C // CB), 0)),
            pl.BlockSpec((CB, 1), lambda i: (i % (C // CB), 0)),
        ],
        out_specs=[
            pl.BlockSpec((CB, H, W), lambda i: (i, 0, 0)),
            pl.BlockSpec((CB, 1), lambda i: (i, 0)),
        ],
        out_shape=[
            jax.ShapeDtypeStruct((B * C, H, W), jnp.float32),
            jax.ShapeDtypeStruct((B * C, 1), jnp.float32),
        ],
    )(xpad, w49, bias)
    return f, pool


def _se_kernel(pool_ref, w1_ref, w2_ref, attn_ref):
    pooled = pool_ref[...] * (1.0 / (H * W))
    h1 = jax.nn.relu(
        lax.dot_general(pooled, w1_ref[...], (((1,), (1,)), ((), ())),
                        preferred_element_type=jnp.float32))
    a = lax.dot_general(h1, w2_ref[...], (((1,), (1,)), ((), ())),
                        preferred_element_type=jnp.float32)
    attn_ref[...] = jax.nn.sigmoid(a)


def _run_se(pool, ca_w1, ca_w2):
    return pl.pallas_call(
        _se_kernel,
        grid=(1,),
        in_specs=[
            pl.BlockSpec((B, C), lambda i: (0, 0)),
            pl.BlockSpec((C // 16, C), lambda i: (0, 0)),
            pl.BlockSpec((C, C // 16), lambda i: (0, 0)),
        ],
        out_specs=pl.BlockSpec((B, C), lambda i: (0, 0)),
        out_shape=jax.ShapeDtypeStruct((B, C), jnp.float32),
    )(pool, ca_w1, ca_w2)


def _mid_kernel(f_ref, attn_ref, g_ref, b_ref, ow_ref,
                x0_ref, y0_ref, wx_ref, wy_ref):
    j = pl.program_id(1)
    f = f_ref[0]
    f = f * attn_ref[0]
    f = 0.5 * f * (1.0 + lax.erf(f * (1.0 / math.sqrt(2.0))))
    mu = jnp.mean(f, axis=0, keepdims=True)
    d = f - mu
    var = jnp.mean(d * d, axis=0, keepdims=True)
    fln = d * lax.rsqrt(var + 1e-5) * g_ref[...] + b_ref[...]
    offx = jnp.sum(fln * ow_ref[:, 0:1], axis=0, keepdims=True)
    offy = jnp.sum(fln * ow_ref[:, 1:2], axis=0, keepdims=True)
    t = jax.lax.broadcasted_iota(jnp.int32, (1, TB3), 1) + j * TB3
    wcol = t % W
    hrow = t // W
    ref_x = -1.0 + wcol.astype(jnp.float32) * (2.0 / (W - 1))
    ref_y = -1.0 + hrow.astype(jnp.float32) * (2.0 / (H - 1))
    gx = ref_x + offx
    gy = ref_y + offy
    ix = jnp.clip(((gx + 1.0) * W - 1.0) * 0.5, 0.0, W - 1.0)
    iy = jnp.clip(((gy + 1.0) * H - 1.0) * 0.5, 0.0, H - 1.0)
    x0 = jnp.floor(ix)
    y0 = jnp.floor(iy)
    x0_ref[0] = x0
    y0_ref[0] = y0
    wx_ref[0] = ix - x0
    wy_ref[0] = iy - y0


def _run_mid(f3, attn3, ln_g, ln_b, off_wT):
    outs = pl.pallas_call(
        _mid_kernel,
        grid=(B, N // TB3),
        in_specs=[
            pl.BlockSpec((1, C, TB3), lambda b, j: (b, 0, j)),
            pl.BlockSpec((1, C, 1), lambda b, j: (b, 0, 0)),
            pl.BlockSpec((C, 1), lambda b, j: (0, 0)),
            pl.BlockSpec((C, 1), lambda b, j: (0, 0)),
            pl.BlockSpec((C, 2), lambda b, j: (0, 0)),
        ],
        out_specs=[pl.BlockSpec((1, 1, TB3), lambda b, j: (b, 0, j))] * 4,
        out_shape=[jax.ShapeDtypeStruct((B, 1, N), jnp.float32)] * 4,
    )(f3, attn3, ln_g, ln_b, off_wT)
    return outs


def _sim_kernel(xn_ref, cn_ref, gi_ref):
    sims = lax.dot_general(xn_ref[...], cn_ref[...], (((1,), (1,)), ((), ())),
                           preferred_element_type=jnp.float32)
    best = sims[:, 0:1]
    bi = jnp.zeros((TB4, 1), jnp.int32)
    for m in range(1, M):
        v = sims[:, m:m + 1]
        upd = v > best
        bi = jnp.where(upd, m, bi)
        best = jnp.where(upd, v, best)
    gi_ref[...] = bi


def _run_sim(xn, cn):
    return pl.pallas_call(
        _sim_kernel,
        grid=(B * N // TB4,),
        in_specs=[pl.BlockSpec((TB4, C), lambda i: (i, 0)),
                  pl.BlockSpec((M, C), lambda i: (0, 0))],
        out_specs=pl.BlockSpec((TB4, 1), lambda i: (i, 0)),
        out_shape=jax.ShapeDtypeStruct((B * N, 1), jnp.int32),
    )(xn, cn)


def kernel(x, dw_w, dw_b, ca_w1, ca_w2, off_w, ln_g, ln_b, R, centers):
    xpad = jnp.pad(x, ((0, 0), (0, 0), (3, 5), (3, 5))).reshape(B * C, 232, 232)
    w49 = dw_w.reshape(C, K * K)
    bias = dw_b.reshape(C, 1)

    f, pool = _run_conv(xpad, w49, bias)
    attn = _run_se(pool.reshape(B, C), ca_w1, ca_w2)

    f3 = f.reshape(B, C, N)
    x0f, y0f, wxf, wyf = _run_mid(f3, attn.reshape(B, C, 1),
                                  ln_g.reshape(C, 1), ln_b.reshape(C, 1),
                                  off_w.T)

    x_seq = lax.stop_gradient(jnp.transpose(x.reshape(B, C, N), (0, 2, 1)))
    xn = _normalize(x_seq)
    cn = _normalize(centers)
    gi = _run_sim(xn.reshape(B * N, C), cn).reshape(B, N)

    perm = jnp.argsort(gi, axis=1)

    x_tok = jnp.transpose(x.reshape(B, C, N), (0, 2, 1))
    x0 = x0f.reshape(B, N).astype(jnp.int32)
    y0 = y0f.reshape(B, N).astype(jnp.int32)
    wx = wxf.reshape(B, N)
    wy = wyf.reshape(B, N)
    x1 = jnp.minimum(x0 + 1, W - 1)
    y1 = jnp.minimum(y0 + 1, H - 1)

    def gat(yi, xi):
        return jnp.take_along_axis(x_tok, (yi * W + xi)[..., None], axis=1)

    v00 = gat(y0, x0)
    v01 = gat(y0, x1)
    v10 = gat(y1, x0)
    v11 = gat(y1, x1)
    wxe = wx[..., None]
    wye = wy[..., None]
    x_bar = (v00 * (1 - wxe) * (1 - wye) + v01 * wxe * (1 - wye)
             + v10 * (1 - wxe) * wye + v11 * wxe * wye)
    permuted = jnp.take_along_axis(x_bar, perm[:, :, None], axis=1)
    out = jnp.transpose(permuted, (0, 2, 1)).reshape(B, C, H, W)
    return out, perm, centers


# speedup vs baseline: 1.0611x; 1.0611x over previous
import functools
import math

import jax
import jax.numpy as jnp
from jax import lax
from jax.experimental import pallas as pl
from jax.experimental.pallas import tpu as pltpu

B, C, H, W = 2, 96, 224, 224
M = 12
K = 7
N = H * W

CB = 8
TB3 = 3584
TB4 = 3584


def _normalize(v, axis=-1):
    n = jnp.linalg.norm(v, axis=axis, keepdims=True)
    return v / jnp.maximum(n, 1e-12)


def _conv_kernel(x_ref, w_ref, b_ref, f_ref, pool_ref):
    xb = x_ref[...].astype(jnp.float32)
    acc = jnp.zeros((CB, H, W), jnp.float32)
    for dy in range(K):
        for dx in range(K):
            k = dy * K + dx
            wch = w_ref[:, k].reshape(CB, 1, 1)
            acc = acc + wch * xb[:, dy:dy + H, dx:dx + W]
    acc = acc + b_ref[:, 0].reshape(CB, 1, 1)
    f_ref[...] = acc
    pool_ref[...] = jnp.sum(acc, axis=(1, 2))[:, None]


def _run_conv(xpad, w49, bias):
    nblk = B * C // CB
    f, pool = pl.pallas_call(
        _conv_kernel,
        grid=(nblk,),
        in_specs=[
            pl.BlockSpec((CB, 232, 232), lambda i: (i, 0, 0)),
            pl.BlockSpec((CB, K * K), lambda i: (i % (C // CB), 0)),
            pl.BlockSpec((CB, 1), lambda i: (i % (C // CB), 0)),
        ],
        out_specs=[
            pl.BlockSpec((CB, H, W), lambda i: (i, 0, 0)),
            pl.BlockSpec((CB, 1), lambda i: (i, 0)),
        ],
        out_shape=[
            jax.ShapeDtypeStruct((B * C, H, W), jnp.float32),
            jax.ShapeDtypeStruct((B * C, 1), jnp.float32),
        ],
    )(xpad, w49, bias)
    return f, pool


def _se_kernel(pool_ref, w1_ref, w2_ref, attn_ref):
    pooled = pool_ref[...] * (1.0 / (H * W))
    h1 = jax.nn.relu(
        lax.dot_general(pooled, w1_ref[...], (((1,), (1,)), ((), ())),
                        preferred_element_type=jnp.float32))
    a = lax.dot_general(h1, w2_ref[...], (((1,), (1,)), ((), ())),
                        preferred_element_type=jnp.float32)
    attn_ref[...] = jax.nn.sigmoid(a)


def _run_se(pool, ca_w1, ca_w2):
    return pl.pallas_call(
        _se_kernel,
        grid=(1,),
        in_specs=[
            pl.BlockSpec((B, C), lambda i: (0, 0)),
            pl.BlockSpec((C // 16, C), lambda i: (0, 0)),
            pl.BlockSpec((C, C // 16), lambda i: (0, 0)),
        ],
        out_specs=pl.BlockSpec((B, C), lambda i: (0, 0)),
        out_shape=jax.ShapeDtypeStruct((B, C), jnp.float32),
    )(pool, ca_w1, ca_w2)


def _mid_kernel(f_ref, attn_ref, g_ref, b_ref, ow_ref,
                x0_ref, y0_ref, wx_ref, wy_ref):
    j = pl.program_id(1)
    f = f_ref[0]
    f = f * attn_ref[0]
    f = 0.5 * f * (1.0 + lax.erf(f * (1.0 / math.sqrt(2.0))))
    mu = jnp.mean(f, axis=0, keepdims=True)
    d = f - mu
    var = jnp.mean(d * d, axis=0, keepdims=True)
    fln = d * lax.rsqrt(var + 1e-5) * g_ref[...] + b_ref[...]
    offs = lax.dot_general(ow_ref[...], fln, (((0,), (0,)), ((), ())),
                           preferred_element_type=jnp.float32)
    offx = offs[0:1]
    offy = offs[1:2]
    t = jax.lax.broadcasted_iota(jnp.int32, (1, TB3), 1) + j * TB3
    wcol = t % W
    hrow = t // W
    ref_x = -1.0 + wcol.astype(jnp.float32) * (2.0 / (W - 1))
    ref_y = -1.0 + hrow.astype(jnp.float32) * (2.0 / (H - 1))
    gx = ref_x + offx
    gy = ref_y + offy
    ix = jnp.clip(((gx + 1.0) * W - 1.0) * 0.5, 0.0, W - 1.0)
    iy = jnp.clip(((gy + 1.0) * H - 1.0) * 0.5, 0.0, H - 1.0)
    x0 = jnp.floor(ix)
    y0 = jnp.floor(iy)
    x0_ref[0] = x0
    y0_ref[0] = y0
    wx_ref[0] = ix - x0
    wy_ref[0] = iy - y0


def _run_mid(f3, attn3, ln_g, ln_b, off_wT):
    outs = pl.pallas_call(
        _mid_kernel,
        grid=(B, N // TB3),
        in_specs=[
            pl.BlockSpec((1, C, TB3), lambda b, j: (b, 0, j)),
            pl.BlockSpec((1, C, 1), lambda b, j: (b, 0, 0)),
            pl.BlockSpec((C, 1), lambda b, j: (0, 0)),
            pl.BlockSpec((C, 1), lambda b, j: (0, 0)),
            pl.BlockSpec((C, 2), lambda b, j: (0, 0)),
        ],
        out_specs=[pl.BlockSpec((1, 1, TB3), lambda b, j: (b, 0, j))] * 4,
        out_shape=[jax.ShapeDtypeStruct((B, 1, N), jnp.float32)] * 4,
    )(f3, attn3, ln_g, ln_b, off_wT)
    return outs


def _sim_kernel(xn_ref, cn_ref, gi_ref):
    sims = lax.dot_general(xn_ref[...], cn_ref[...], (((1,), (1,)), ((), ())),
                           preferred_element_type=jnp.float32)
    best = sims[:, 0:1]
    bi = jnp.zeros((TB4, 1), jnp.int32)
    for m in range(1, M):
        v = sims[:, m:m + 1]
        upd = v > best
        bi = jnp.where(upd, m, bi)
        best = jnp.where(upd, v, best)
    gi_ref[...] = bi


def _run_sim(xn, cn):
    return pl.pallas_call(
        _sim_kernel,
        grid=(B * N // TB4,),
        in_specs=[pl.BlockSpec((TB4, C), lambda i: (i, 0)),
                  pl.BlockSpec((M, C), lambda i: (0, 0))],
        out_specs=pl.BlockSpec((TB4, 1), lambda i: (i, 0)),
        out_shape=jax.ShapeDtypeStruct((B * N, 1), jnp.int32),
    )(xn, cn)


def kernel(x, dw_w, dw_b, ca_w1, ca_w2, off_w, ln_g, ln_b, R, centers):
    xpad = jnp.pad(x, ((0, 0), (0, 0), (3, 5), (3, 5))).reshape(
        B * C, 232, 232).astype(jnp.bfloat16)
    w49 = dw_w.reshape(C, K * K)
    bias = dw_b.reshape(C, 1)

    f, pool = _run_conv(xpad, w49, bias)
    attn = _run_se(pool.reshape(B, C), ca_w1, ca_w2)

    f3 = f.reshape(B, C, N)
    x0f, y0f, wxf, wyf = _run_mid(f3, attn.reshape(B, C, 1),
                                  ln_g.reshape(C, 1), ln_b.reshape(C, 1),
                                  off_w.T)

    x_seq = lax.stop_gradient(jnp.transpose(x.reshape(B, C, N), (0, 2, 1)))
    xn = _normalize(x_seq)
    cn = _normalize(centers)
    gi = _run_sim(xn.reshape(B * N, C), cn).reshape(B, N)

    perm = jnp.argsort(gi, axis=1)

    x_tok = jnp.transpose(x.reshape(B, C, N), (0, 2, 1))
    x0 = x0f.reshape(B, N).astype(jnp.int32)
    y0 = y0f.reshape(B, N).astype(jnp.int32)
    wx = wxf.reshape(B, N)
    wy = wyf.reshape(B, N)
    x1 = jnp.minimum(x0 + 1, W - 1)
    y1 = jnp.minimum(y0 + 1, H - 1)

    def gat(yi, xi):
        return jnp.take_along_axis(x_tok, (yi * W + xi)[..., None], axis=1)

    v00 = gat(y0, x0)
    v01 = gat(y0, x1)
    v10 = gat(y1, x0)
    v11 = gat(y1, x1)
    wxe = wx[..., None]
    wye = wy[..., None]
    x_bar = (v00 * (1 - wxe) * (1 - wye) + v01 * wxe * (1 - wye)
             + v10 * (1 - wxe) * wye + v11 * wxe * wye)
    permuted = jnp.take_along_axis(x_bar, perm[:, :, None], axis=1)
    out = jnp.transpose(permuted, (0, 2, 1)).reshape(B, C, H, W)
    return out, perm, centers


# speedup vs baseline: 1.2077x; 1.1382x over previous
import functools
import math

import jax
import jax.numpy as jnp
from jax import lax
from jax.experimental import pallas as pl
from jax.experimental.pallas import tpu as pltpu
from jax.experimental.pallas import tpu_sc as plsc

B, C, H, W = 2, 96, 224, 224
M = 12
K = 7
N = H * W

CB = 8
TB3 = 3584
TB4 = 3584


def _normalize(v, axis=-1):
    n = jnp.linalg.norm(v, axis=axis, keepdims=True)
    return v / jnp.maximum(n, 1e-12)


def _conv_kernel(x_ref, w_ref, b_ref, f_ref, pool_ref):
    xb = x_ref[...].astype(jnp.float32)
    acc = jnp.zeros((CB, H, W), jnp.float32)
    for dy in range(K):
        for dx in range(K):
            k = dy * K + dx
            wch = w_ref[:, k].reshape(CB, 1, 1)
            acc = acc + wch * xb[:, dy:dy + H, dx:dx + W]
    acc = acc + b_ref[:, 0].reshape(CB, 1, 1)
    f_ref[...] = acc
    pool_ref[...] = jnp.sum(acc, axis=(1, 2))[:, None]


def _run_conv(xpad, w49, bias):
    nblk = B * C // CB
    f, pool = pl.pallas_call(
        _conv_kernel,
        grid=(nblk,),
        in_specs=[
            pl.BlockSpec((CB, 232, 232), lambda i: (i, 0, 0)),
            pl.BlockSpec((CB, K * K), lambda i: (i % (C // CB), 0)),
            pl.BlockSpec((CB, 1), lambda i: (i % (C // CB), 0)),
        ],
        out_specs=[
            pl.BlockSpec((CB, H, W), lambda i: (i, 0, 0)),
            pl.BlockSpec((CB, 1), lambda i: (i, 0)),
        ],
        out_shape=[
            jax.ShapeDtypeStruct((B * C, H, W), jnp.float32),
            jax.ShapeDtypeStruct((B * C, 1), jnp.float32),
        ],
    )(xpad, w49, bias)
    return f, pool


def _se_kernel(pool_ref, w1_ref, w2_ref, attn_ref):
    pooled = pool_ref[...] * (1.0 / (H * W))
    h1 = jax.nn.relu(
        lax.dot_general(pooled, w1_ref[...], (((1,), (1,)), ((), ())),
                        preferred_element_type=jnp.float32))
    a = lax.dot_general(h1, w2_ref[...], (((1,), (1,)), ((), ())),
                        preferred_element_type=jnp.float32)
    attn_ref[...] = jax.nn.sigmoid(a)


def _run_se(pool, ca_w1, ca_w2):
    return pl.pallas_call(
        _se_kernel,
        grid=(1,),
        in_specs=[
            pl.BlockSpec((B, C), lambda i: (0, 0)),
            pl.BlockSpec((C // 16, C), lambda i: (0, 0)),
            pl.BlockSpec((C, C // 16), lambda i: (0, 0)),
        ],
        out_specs=pl.BlockSpec((B, C), lambda i: (0, 0)),
        out_shape=jax.ShapeDtypeStruct((B, C), jnp.float32),
    )(pool, ca_w1, ca_w2)


def _mid_kernel(f_ref, attn_ref, g_ref, b_ref, ow_ref,
                x0_ref, y0_ref, wx_ref, wy_ref):
    j = pl.program_id(1)
    f = f_ref[0]
    f = f * attn_ref[0]
    f = 0.5 * f * (1.0 + lax.erf(f * (1.0 / math.sqrt(2.0))))
    mu = jnp.mean(f, axis=0, keepdims=True)
    d = f - mu
    var = jnp.mean(d * d, axis=0, keepdims=True)
    fln = d * lax.rsqrt(var + 1e-5) * g_ref[...] + b_ref[...]
    offs = lax.dot_general(ow_ref[...], fln, (((0,), (0,)), ((), ())),
                           preferred_element_type=jnp.float32)
    offx = offs[0:1]
    offy = offs[1:2]
    t = jax.lax.broadcasted_iota(jnp.int32, (1, TB3), 1) + j * TB3
    wcol = t % W
    hrow = t // W
    ref_x = -1.0 + wcol.astype(jnp.float32) * (2.0 / (W - 1))
    ref_y = -1.0 + hrow.astype(jnp.float32) * (2.0 / (H - 1))
    gx = ref_x + offx
    gy = ref_y + offy
    ix = jnp.clip(((gx + 1.0) * W - 1.0) * 0.5, 0.0, W - 1.0)
    iy = jnp.clip(((gy + 1.0) * H - 1.0) * 0.5, 0.0, H - 1.0)
    x0 = jnp.floor(ix)
    y0 = jnp.floor(iy)
    x0_ref[0] = x0
    y0_ref[0] = y0
    wx_ref[0] = ix - x0
    wy_ref[0] = iy - y0


def _run_mid(f3, attn3, ln_g, ln_b, off_wT):
    outs = pl.pallas_call(
        _mid_kernel,
        grid=(B, N // TB3),
        in_specs=[
            pl.BlockSpec((1, C, TB3), lambda b, j: (b, 0, j)),
            pl.BlockSpec((1, C, 1), lambda b, j: (b, 0, 0)),
            pl.BlockSpec((C, 1), lambda b, j: (0, 0)),
            pl.BlockSpec((C, 1), lambda b, j: (0, 0)),
            pl.BlockSpec((C, 2), lambda b, j: (0, 0)),
        ],
        out_specs=[pl.BlockSpec((1, 1, TB3), lambda b, j: (b, 0, j))] * 4,
        out_shape=[jax.ShapeDtypeStruct((B, 1, N), jnp.float32)] * 4,
    )(f3, attn3, ln_g, ln_b, off_wT)
    return outs


def _sim_kernel(xn_ref, cn_ref, gi_ref):
    sims = lax.dot_general(xn_ref[...], cn_ref[...], (((1,), (1,)), ((), ())),
                           preferred_element_type=jnp.float32)
    best = sims[:, 0:1]
    bi = jnp.zeros((TB4, 1), jnp.int32)
    for m in range(1, M):
        v = sims[:, m:m + 1]
        upd = v > best
        bi = jnp.where(upd, m, bi)
        best = jnp.where(upd, v, best)
    gi_ref[...] = bi


def _run_sim(xn, cn):
    return pl.pallas_call(
        _sim_kernel,
        grid=(B * N // TB4,),
        in_specs=[pl.BlockSpec((TB4, C), lambda i: (i, 0)),
                  pl.BlockSpec((M, C), lambda i: (0, 0))],
        out_specs=pl.BlockSpec((TB4, 1), lambda i: (i, 0)),
        out_shape=jax.ShapeDtypeStruct((B * N, 1), jnp.int32),
    )(xn, cn)


NT = 16
CH = N // NT
KW = 112
NV = CH // 16
NW = CH // KW
CP = 128


def _sc_body(gi_hbm, x0_hbm, y0_hbm, wx_hbm, wy_hbm, x_hbm,
             perm_hbm, out_hbm,
             gi_v, pos_v, val_v, hist_v, grid_v,
             gidx_v, p0_v, p1_v, p2_v, p3_v,
             i00, i01, i10, i11, w00, w01, w10, w11,
             v00, v01, v10, v11, outw, hist_sp, perm_sp, sem):
    ci = lax.axis_index("c")
    si = lax.axis_index("s")
    bN = ci * N
    cbase = bN + si * CH
    lane = lax.iota(jnp.int32, 16)

    pltpu.sync_copy(gi_hbm.at[pl.ds(cbase, CH)], gi_v)

    def hbody(i, hist):
        g = gi_v[pl.ds(i * 16, 16)]
        for m in range(M):
            pc = plsc.all_reduce_population_count(g == m)
            hist = hist + jnp.where(lane == m, pc, 0)
        return hist

    hist_v[...] = lax.fori_loop(0, NV, hbody, jnp.zeros((16,), jnp.int32))
    pltpu.sync_copy(hist_v, hist_sp.at[si])
    plsc.subcore_barrier()

    pltpu.sync_copy(hist_sp, grid_v)
    totals = jnp.zeros((16,), jnp.int32)
    pre = jnp.zeros((16,), jnp.int32)
    for s in range(NT):
        row = grid_v[s]
        totals = totals + row
        pre = pre + jnp.where(jnp.broadcast_to(s < si, (16,)), row, 0)
    gbase = plsc.cumsum(totals) - totals
    start0 = gbase + pre

    def cbody(i, start):
        g = gi_v[pl.ds(i * 16, 16)]
        pos = jnp.zeros((16,), jnp.int32)
        for m in range(M):
            msk = g == m
            incl = plsc.cumsum(msk.astype(jnp.int32))
            bm = jnp.take(start, jnp.full((16,), m, jnp.int32))
            pos = jnp.where(msk, bm + incl - 1, pos)
            pc = plsc.all_reduce_population_count(msk)
            start = jnp.where(lane == m, start + pc, start)
        pos_v[pl.ds(i * 16, 16)] = jnp.clip(pos, 0, N - 1)
        val_v[pl.ds(i * 16, 16)] = (si * CH + i * 16) + lane
        return start

    lax.fori_loop(0, NV, cbody, start0)
    pltpu.async_copy(val_v, perm_sp.at[pos_v], sem).wait()
    plsc.subcore_barrier()
    pltpu.sync_copy(perm_sp.at[pl.ds(si * CH, CH)], gi_v)
    pltpu.sync_copy(gi_v, perm_hbm.at[pl.ds(cbase, CH)])


    def wbody(w, carry):
        wb = w * KW
        for i in range(KW // 16):
            gidx_v[pl.ds(i * 16, 16)] = jnp.clip(
                gi_v[pl.ds(wb + i * 16, 16)] + bN, 0, B * N - 1)
        pltpu.async_copy(x0_hbm.at[gidx_v], p0_v, sem).wait()
        pltpu.async_copy(y0_hbm.at[gidx_v], p1_v, sem).wait()
        pltpu.async_copy(wx_hbm.at[gidx_v], p2_v, sem).wait()
        pltpu.async_copy(wy_hbm.at[gidx_v], p3_v, sem).wait()
        for i in range(KW // 16):
            sl = pl.ds(i * 16, 16)
            xi0 = p0_v[sl].astype(jnp.int32)
            yi0 = p1_v[sl].astype(jnp.int32)
            wx = p2_v[sl]
            wy = p3_v[sl]
            xi1 = jnp.minimum(xi0 + 1, W - 1)
            yi1 = jnp.minimum(yi0 + 1, H - 1)
            hi = B * N - 1
            i00[sl] = jnp.clip(bN + yi0 * W + xi0, 0, hi)
            i01[sl] = jnp.clip(bN + yi0 * W + xi1, 0, hi)
            i10[sl] = jnp.clip(bN + yi1 * W + xi0, 0, hi)
            i11[sl] = jnp.clip(bN + yi1 * W + xi1, 0, hi)
            w00[sl] = (1.0 - wx) * (1.0 - wy)
            w01[sl] = wx * (1.0 - wy)
            w10[sl] = (1.0 - wx) * wy
            w11[sl] = wx * wy
        pltpu.async_copy(x_hbm.at[i00], v00, sem).wait()
        pltpu.async_copy(x_hbm.at[i01], v01, sem).wait()
        pltpu.async_copy(x_hbm.at[i10], v10, sem).wait()
        pltpu.async_copy(x_hbm.at[i11], v11, sem).wait()

        def jbody(j, c2):
            jf = jnp.full((16,), j, jnp.int32)
            a = plsc.load_gather(w00, [jf])
            b_ = plsc.load_gather(w01, [jf])
            c_ = plsc.load_gather(w10, [jf])
            d_ = plsc.load_gather(w11, [jf])
            for l in range(C // 16):
                s2 = pl.ds(l * 16, 16)
                outw[j, s2] = (v00[j, s2] * a + v01[j, s2] * b_
                               + v10[j, s2] * c_ + v11[j, s2] * d_)
            return c2

        lax.fori_loop(0, KW, jbody, 0)
        pltpu.sync_copy(outw, out_hbm.at[pl.ds(cbase + wb, KW)])
        return carry

    lax.fori_loop(0, NW, wbody, 0)


def _run_sc(gi_flat, x0p, y0p, wxp, wyp, xp):
    mesh = plsc.VectorSubcoreMesh(core_axis_name="c", subcore_axis_name="s")
    fn = functools.partial(
        pl.kernel,
        mesh=mesh,
        out_type=[jax.ShapeDtypeStruct((B * N,), jnp.int32),
                  jax.ShapeDtypeStruct((B * N, CP), jnp.float32)],
        compiler_params=pltpu.CompilerParams(needs_layout_passes=False),
        scratch_types=[
            pltpu.VMEM((CH,), jnp.int32),
            pltpu.VMEM((CH,), jnp.int32),
            pltpu.VMEM((CH,), jnp.int32),
            pltpu.VMEM((16,), jnp.int32),
            pltpu.VMEM((NT, 16), jnp.int32),
            pltpu.VMEM((KW,), jnp.int32),
            pltpu.VMEM((KW,), jnp.float32),
            pltpu.VMEM((KW,), jnp.float32),
            pltpu.VMEM((KW,), jnp.float32),
            pltpu.VMEM((KW,), jnp.float32),
            pltpu.VMEM((KW,), jnp.int32),
            pltpu.VMEM((KW,), jnp.int32),
            pltpu.VMEM((KW,), jnp.int32),
            pltpu.VMEM((KW,), jnp.int32),
            pltpu.VMEM((KW,), jnp.float32),
            pltpu.VMEM((KW,), jnp.float32),
            pltpu.VMEM((KW,), jnp.float32),
            pltpu.VMEM((KW,), jnp.float32),
            pltpu.VMEM((KW, CP), jnp.float32),
            pltpu.VMEM((KW, CP), jnp.float32),
            pltpu.VMEM((KW, CP), jnp.float32),
            pltpu.VMEM((KW, CP), jnp.float32),
            pltpu.VMEM((KW, CP), jnp.float32),
            pltpu.VMEM_SHARED((NT, 16), jnp.int32),
            pltpu.VMEM_SHARED((N,), jnp.int32),
            pltpu.SemaphoreType.DMA,
        ],
    )(_sc_body)
    return fn(gi_flat, x0p, y0p, wxp, wyp, xp)


def kernel(x, dw_w, dw_b, ca_w1, ca_w2, off_w, ln_g, ln_b, R, centers):
    xpad = jnp.pad(x, ((0, 0), (0, 0), (3, 5), (3, 5))).reshape(
        B * C, 232, 232).astype(jnp.bfloat16)
    w49 = dw_w.reshape(C, K * K)
    bias = dw_b.reshape(C, 1)

    f, pool = _run_conv(xpad, w49, bias)
    attn = _run_se(pool.reshape(B, C), ca_w1, ca_w2)

    f3 = f.reshape(B, C, N)
    x0f, y0f, wxf, wyf = _run_mid(f3, attn.reshape(B, C, 1),
                                  ln_g.reshape(C, 1), ln_b.reshape(C, 1),
                                  off_w.T)

    x_seq = lax.stop_gradient(jnp.transpose(x.reshape(B, C, N), (0, 2, 1)))
    xn = _normalize(x_seq)
    cn = _normalize(centers)
    gi = _run_sim(xn.reshape(B * N, C), cn)

    x_flat = jnp.transpose(x.reshape(B, C, N), (0, 2, 1)).reshape(B * N, C)
    xp = jnp.pad(x_flat, ((0, 0), (0, CP - C)))

    perm_flat, out_tok = _run_sc(gi.reshape(B * N),
                                 x0f.reshape(B * N), y0f.reshape(B * N),
                                 wxf.reshape(B * N), wyf.reshape(B * N), xp)
    perm = perm_flat.reshape(B, N)
    out = jnp.transpose(out_tok[:, :C].reshape(B, N, C),
                        (0, 2, 1)).reshape(B, C, H, W)
    return out, perm, centers
